# Initial kernel scaffold; baseline (speedup 1.0000x reference)
import functools

import jax
import jax.numpy as jnp
from jax import lax
from jax.experimental import pallas as pl
from jax.experimental.pallas import tpu as pltpu
from jax.experimental.pallas import tpu_sc as plsc

N = 10000
E = 320000
D = 128
H = 32

NC = 2
NS = 16
NW = NC * NS
L = 16

EW = E // NW
C = 400
NCH = EW // C
G = 80
RPC = C // G
NSTR = N // NS
ZR = 125


def _nodeproj_body(x_ref, wl_ref, wr_ref, bl_ref, br_ref, xl_ref, xr_ref):
  x = x_ref[...]
  xl_ref[...] = jnp.dot(x, wl_ref[...], preferred_element_type=jnp.float32) + bl_ref[...]
  xr_ref[...] = jnp.dot(x, wr_ref[...], preferred_element_type=jnp.float32) + br_ref[...]


def _node_proj(x, wl, wr, bl, br):
  return pl.pallas_call(
      _nodeproj_body,
      out_shape=(jax.ShapeDtypeStruct((N, H), jnp.float32),
                 jax.ShapeDtypeStruct((N, H), jnp.float32)),
  )(x, wl, wr, bl.reshape(1, H), br.reshape(1, H))


def _edgeproj_body(ea_ref, we_ref, out_ref):
  out_ref[...] = jnp.dot(ea_ref[...], we_ref[...],
                         preferred_element_type=jnp.float32)


def _edge_proj(ea, we):
  de = ea.shape[1]
  blk = 2500
  return pl.pallas_call(
      _edgeproj_body,
      grid=(E // blk,),
      in_specs=[pl.BlockSpec((blk, de), lambda i: (i, 0)),
                pl.BlockSpec((de, H), lambda i: (0, 0))],
      out_specs=pl.BlockSpec((blk, H), lambda i: (i, 0)),
      out_shape=jax.ShapeDtypeStruct((E, H), jnp.float32),
  )(ea, we)


def _edgemul_body(a_ref, we_ref, out_ref):
  out_ref[...] = a_ref[...] * we_ref[...]


def _edge_mul(a1, we):
  blk = 2500
  return pl.pallas_call(
      _edgemul_body,
      grid=(E // blk,),
      in_specs=[pl.BlockSpec((blk, 1), lambda i: (i, 0)),
                pl.BlockSpec((1, H), lambda i: (0, 0))],
      out_specs=pl.BlockSpec((blk, H), lambda i: (i, 0)),
      out_shape=jax.ShapeDtypeStruct((E, H), jnp.float32),
  )(a1, we)


def _denom_body(dpart_ref, out_ref):
  s = jnp.sum(dpart_ref[...], axis=0, keepdims=True)
  out_ref[...] = 1.0 / (s + 1e-16)


def _denom_combine(dpart):
  return pl.pallas_call(
      _denom_body,
      out_shape=jax.ShapeDtypeStruct((1, N), jnp.float32),
  )(dpart)


def _assemble_body(op_ref, b_ref, wl_ref, wr_ref, bl_ref, br_ref,
                   xl_ref, xr_ref):
  h = jax.nn.relu(op_ref[0] + op_ref[1] + b_ref[...])
  xl_ref[...] = jnp.dot(h, wl_ref[...], preferred_element_type=jnp.float32) + bl_ref[...]
  xr_ref[...] = jnp.dot(h, wr_ref[...], preferred_element_type=jnp.float32) + br_ref[...]


def _assemble_proj(opart, bias, wl, wr, bl, br):
  return pl.pallas_call(
      _assemble_body,
      out_shape=(jax.ShapeDtypeStruct((N, H), jnp.float32),
                 jax.ShapeDtypeStruct((N, H), jnp.float32)),
  )(opart, bias.reshape(1, H), wl, wr, bl.reshape(1, H), br.reshape(1, H))


def _final_body(op_ref, b_ref, out_ref):
  out_ref[...] = op_ref[0] + op_ref[1] + b_ref[...]


def _final_assemble(opart, bias):
  return pl.pallas_call(
      _final_body,
      out_shape=jax.ShapeDtypeStruct((N, H), jnp.float32),
  )(opart, bias.reshape(1, H))


def _sc_mesh():
  return plsc.VectorSubcoreMesh(core_axis_name="c", subcore_axis_name="s",
                                num_cores=NC, num_subcores=NS)


def _pass_a_body(src_hbm, dst_hbm, xl_hbm, xr_hbm, eap_hbm, att_hbm,
                 p_hbm, dpart_hbm,
                 idx_s, idx_d, xi_v, xj_v, ea_v, att_v, logit_v, pbuf_v,
                 denom_v, gsem):
  cid = lax.axis_index("c")
  sid = lax.axis_index("s")
  wid = sid * NC + cid
  e0 = wid * EW
  row0 = wid * (EW // G)

  pltpu.sync_copy(att_hbm, att_v)
  att_lo = att_v[pl.ds(0, L)]
  att_hi = att_v[pl.ds(L, L)]
  zero = jnp.zeros((L,), jnp.float32)

  def zero_body(i, c):
    denom_v[pl.ds(i * L, L)] = zero
    return c
  lax.fori_loop(0, N // L, zero_body, 0)

  def chunk_body(ch, c):
    r0 = row0 + ch * RPC
    pltpu.sync_copy(src_hbm.at[pl.ds(r0, RPC)], idx_s)
    pltpu.sync_copy(dst_hbm.at[pl.ds(r0, RPC)], idx_d)
    pltpu.sync_copy(eap_hbm.at[pl.ds(e0 + ch * C, C)], ea_v)
    descs = []
    for j in range(RPC):
      descs.append(pltpu.make_async_copy(
          xl_hbm.at[idx_s.at[j]], xi_v.at[pl.ds(j * G, G)], gsem))
      descs.append(pltpu.make_async_copy(
          xr_hbm.at[idx_d.at[j]], xj_v.at[pl.ds(j * G, G)], gsem))
    for d in descs:
      d.start()
    for d in descs:
      d.wait()

    def edge_body(e, c2):
      va = xi_v[e, pl.ds(0, L)] + xj_v[e, pl.ds(0, L)] + ea_v[e, pl.ds(0, L)]
      vb = xi_v[e, pl.ds(L, L)] + xj_v[e, pl.ds(L, L)] + ea_v[e, pl.ds(L, L)]
      va = jnp.maximum(va, va * 0.2)
      vb = jnp.maximum(vb, vb * 0.2)
      t = va * att_lo + vb * att_hi
      logit_v[e] = jnp.sum(t)
      return c2
    lax.fori_loop(0, C, edge_body, 0)

    for g in range(C // L):
      pv = jnp.exp(logit_v[pl.ds(g * L, L)])
      pbuf_v[pl.ds(ch * C + g * L, L)] = pv
      dd = idx_d[g // (G // L), pl.ds((g % (G // L)) * L, L)]
      plsc.addupdate_scatter(denom_v, [dd], pv)
    return c
  lax.fori_loop(0, NCH, chunk_body, 0)

  pltpu.sync_copy(pbuf_v, p_hbm.at[pl.ds(e0, EW)])
  pltpu.sync_copy(denom_v, dpart_hbm.at[wid])


def _sc_pass_a(src2d, dst2d, xl, xr, eap, att):
  kfn = pl.kernel(
      _pass_a_body,
      out_type=(jax.ShapeDtypeStruct((E,), jnp.float32),
                jax.ShapeDtypeStruct((NW, N), jnp.float32)),
      mesh=_sc_mesh(),
      scratch_types=[
          pltpu.VMEM((RPC, G), jnp.int32),
          pltpu.VMEM((RPC, G), jnp.int32),
          pltpu.VMEM((C, H), jnp.float32),
          pltpu.VMEM((C, H), jnp.float32),
          pltpu.VMEM((C, H), jnp.float32),
          pltpu.VMEM((H,), jnp.float32),
          pltpu.VMEM((C,), jnp.float32),
          pltpu.VMEM((EW,), jnp.float32),
          pltpu.VMEM((N,), jnp.float32),
          pltpu.SemaphoreType.DMA,
      ],
  )
  return kfn(src2d, dst2d, xl, xr, eap, att)


def _pass_b_body(src_hbm, dst_hbm, xl_hbm, p_hbm, rden_hbm,
                 alpha_hbm, opart_hbm,
                 idx_s, idx_d, xj_v, p_v, rden_v, abuf_v, zbuf_v,
                 acc_shared, gsem):
  cid = lax.axis_index("c")
  sid = lax.axis_index("s")
  wid = sid * NC + cid
  e0 = wid * EW
  row0 = wid * (EW // G)

  pltpu.sync_copy(rden_hbm, rden_v)

  zero = jnp.zeros((L,), jnp.float32)

  def zrow(i, c):
    zbuf_v[i, pl.ds(0, L)] = zero
    zbuf_v[i, pl.ds(L, L)] = zero
    return c
  lax.fori_loop(0, ZR, zrow, 0)
  for k in range(NSTR // ZR):
    pltpu.sync_copy(zbuf_v, acc_shared.at[pl.ds(sid * NSTR + k * ZR, ZR)])
  plsc.subcore_barrier()

  def chunk_body(ch, c):
    r0 = row0 + ch * RPC
    pltpu.sync_copy(src_hbm.at[pl.ds(r0, RPC)], idx_s)
    pltpu.sync_copy(dst_hbm.at[pl.ds(r0, RPC)], idx_d)
    pltpu.sync_copy(p_hbm.at[pl.ds(e0 + ch * C, C)], p_v)
    descs = []
    for j in range(RPC):
      descs.append(pltpu.make_async_copy(
          xl_hbm.at[idx_s.at[j]], xj_v.at[pl.ds(j * G, G)], gsem))
    for d in descs:
      d.start()
    for d in descs:
      d.wait()

    for g in range(C // L):
      dd = idx_d[g // (G // L), pl.ds((g % (G // L)) * L, L)]
      dv = plsc.load_gather(rden_v, [dd])
      al = p_v[pl.ds(g * L, L)] * dv
      abuf_v[pl.ds(ch * C + g * L, L)] = al

    def edge_body(e, c2):
      a_s = abuf_v[ch * C + e]
      xj_v[e, pl.ds(0, L)] = xj_v[e, pl.ds(0, L)] * a_s
      xj_v[e, pl.ds(L, L)] = xj_v[e, pl.ds(L, L)] * a_s
      return c2
    lax.fori_loop(0, C, edge_body, 0)

    for j in range(RPC):
      pltpu.sync_copy(xj_v.at[pl.ds(j * G, G)], acc_shared.at[idx_d.at[j]],
                      add=True)
    return c
  lax.fori_loop(0, NCH, chunk_body, 0)

  plsc.subcore_barrier()
  pltpu.sync_copy(acc_shared.at[pl.ds(sid * NSTR, NSTR)],
                  opart_hbm.at[cid, pl.ds(sid * NSTR, NSTR)])
  pltpu.sync_copy(abuf_v, alpha_hbm.at[pl.ds(e0, EW)])


def _sc_pass_b(src2d, dst2d, xl, p, rden):
  kfn = pl.kernel(
      _pass_b_body,
      out_type=(jax.ShapeDtypeStruct((E,), jnp.float32),
                jax.ShapeDtypeStruct((NC, N, H), jnp.float32)),
      mesh=_sc_mesh(),
      scratch_types=[
          pltpu.VMEM((RPC, G), jnp.int32),
          pltpu.VMEM((RPC, G), jnp.int32),
          pltpu.VMEM((C, H), jnp.float32),
          pltpu.VMEM((C,), jnp.float32),
          pltpu.VMEM((N,), jnp.float32),
          pltpu.VMEM((EW,), jnp.float32),
          pltpu.VMEM((ZR, H), jnp.float32),
          pltpu.VMEM_SHARED((N, H), jnp.float32),
          pltpu.SemaphoreType.DMA,
      ],
  )
  return kfn(src2d, dst2d, xl, p, rden)


def kernel(x, edge_index, edge_attr,
           W1l, W1r, b1l, b1r, att1, We1, bias1,
           W2l, W2r, b2l, b2r, att2, We2, bias2):
  src2d = edge_index[0].reshape(E // G, G)
  dst2d = edge_index[1].reshape(E // G, G)

  xl1, xr1 = _node_proj(x, W1l, W1r, b1l, b1r)
  eap1 = _edge_proj(edge_attr, We1)
  p1, dpart1 = _sc_pass_a(src2d, dst2d, xl1, xr1, eap1, att1)
  rden1 = _denom_combine(dpart1).reshape(N)
  a1, opart1 = _sc_pass_b(src2d, dst2d, xl1, p1, rden1)

  xl2, xr2 = _assemble_proj(opart1, bias1, W2l, W2r, b2l, b2r)
  eap2 = _edge_mul(a1.reshape(E, 1), We2)
  p2, dpart2 = _sc_pass_a(src2d, dst2d, xl2, xr2, eap2, att2)
  rden2 = _denom_combine(dpart2).reshape(N)
  a2, opart2 = _sc_pass_b(src2d, dst2d, xl2, p2, rden2)

  x2 = _final_assemble(opart2, bias2)
  return (x2, edge_index, a2)


# speedup vs baseline: 7.9440x; 7.9440x over previous
import functools

import jax
import jax.numpy as jnp
from jax import lax
from jax.experimental import pallas as pl
from jax.experimental.pallas import tpu as pltpu
from jax.experimental.pallas import tpu_sc as plsc

N = 10000
E = 320000
D = 128
H = 32

NC = 2
NS = 16
NW = NC * NS
L = 16

EW = E // NW
C = 400
NCH = EW // C
G = 80
RPC = C // G
STR = 624
TAIL = N - NS * STR
ZB = 104


def _nodeproj_body(x_ref, wl_ref, wr_ref, bl_ref, br_ref, xl_ref, xr_ref):
  x = x_ref[...]
  xl_ref[...] = jnp.dot(x, wl_ref[...], preferred_element_type=jnp.float32) + bl_ref[...]
  xr_ref[...] = jnp.dot(x, wr_ref[...], preferred_element_type=jnp.float32) + br_ref[...]


def _node_proj(x, wl, wr, bl, br):
  return pl.pallas_call(
      _nodeproj_body,
      out_shape=(jax.ShapeDtypeStruct((N, H), jnp.float32),
                 jax.ShapeDtypeStruct((N, H), jnp.float32)),
  )(x, wl, wr, bl.reshape(1, H), br.reshape(1, H))


def _edgeproj_body(ea_ref, we_ref, out_ref):
  out_ref[...] = jnp.dot(ea_ref[...], we_ref[...],
                         preferred_element_type=jnp.float32)


def _edge_proj(ea, we):
  de = ea.shape[1]
  blk = 2000
  return pl.pallas_call(
      _edgeproj_body,
      grid=(E // blk,),
      in_specs=[pl.BlockSpec((blk, de), lambda i: (i, 0)),
                pl.BlockSpec((de, H), lambda i: (0, 0))],
      out_specs=pl.BlockSpec((blk, H), lambda i: (i, 0)),
      out_shape=jax.ShapeDtypeStruct((E, H), jnp.float32),
  )(ea, we)


def _edgemul_body(a_ref, we_ref, out_ref):
  out_ref[...] = a_ref[...] * we_ref[...]


def _edge_mul(a1, we):
  blk = 2000
  return pl.pallas_call(
      _edgemul_body,
      grid=(E // blk,),
      in_specs=[pl.BlockSpec((blk, 1), lambda i: (i, 0)),
                pl.BlockSpec((1, H), lambda i: (0, 0))],
      out_specs=pl.BlockSpec((blk, H), lambda i: (i, 0)),
      out_shape=jax.ShapeDtypeStruct((E, H), jnp.float32),
  )(a1, we)


def _denom_body(dpart_ref, out_ref):
  s = jnp.sum(dpart_ref[...], axis=0, keepdims=True)
  out_ref[...] = 1.0 / (s + 1e-16)


def _denom_combine(dpart):
  return pl.pallas_call(
      _denom_body,
      out_shape=jax.ShapeDtypeStruct((1, N), jnp.float32),
  )(dpart)


def _assemble_body(op_ref, b_ref, wl_ref, wr_ref, bl_ref, br_ref,
                   xl_ref, xr_ref):
  h = jax.nn.relu(op_ref[0] + op_ref[1] + b_ref[...])
  xl_ref[...] = jnp.dot(h, wl_ref[...], preferred_element_type=jnp.float32) + bl_ref[...]
  xr_ref[...] = jnp.dot(h, wr_ref[...], preferred_element_type=jnp.float32) + br_ref[...]


def _assemble_proj(opart, bias, wl, wr, bl, br):
  return pl.pallas_call(
      _assemble_body,
      out_shape=(jax.ShapeDtypeStruct((N, H), jnp.float32),
                 jax.ShapeDtypeStruct((N, H), jnp.float32)),
  )(opart, bias.reshape(1, H), wl, wr, bl.reshape(1, H), br.reshape(1, H))


def _final_body(op_ref, b_ref, out_ref):
  out_ref[...] = op_ref[0] + op_ref[1] + b_ref[...]


def _final_assemble(opart, bias):
  return pl.pallas_call(
      _final_body,
      out_shape=jax.ShapeDtypeStruct((N, H), jnp.float32),
  )(opart, bias.reshape(1, H))


def _sc_mesh():
  return plsc.VectorSubcoreMesh(core_axis_name="c", subcore_axis_name="s",
                                num_cores=NC, num_subcores=NS)


def _pass_a_body(src_hbm, dst_hbm, xl_hbm, xr_hbm, eap_hbm, att_hbm,
                 p_hbm, dpart_hbm,
                 idx_s, idx_d, xi_v, xj_v, ea_v, att_v, logit_v, pbuf_v,
                 denom_v, gsem):
  cid = lax.axis_index("c")
  sid = lax.axis_index("s")
  wid = sid * NC + cid
  e0 = wid * EW

  pltpu.sync_copy(att_hbm, att_v)
  att_lo = att_v[pl.ds(0, L)]
  att_hi = att_v[pl.ds(L, L)]
  zero = jnp.zeros((L,), jnp.float32)

  def zero_body(i, c):
    denom_v[pl.ds(i * L, L)] = zero
    return c
  lax.fori_loop(0, N // L, zero_body, 0)

  def chunk_body(ch, c):
    pltpu.sync_copy(src_hbm.at[wid, ch], idx_s)
    pltpu.sync_copy(dst_hbm.at[wid, ch], idx_d)
    pltpu.sync_copy(eap_hbm.at[pl.ds(e0 + ch * C, C)], ea_v)
    descs = []
    for j in range(RPC):
      descs.append(pltpu.make_async_copy(
          xl_hbm.at[idx_s.at[j]], xi_v.at[pl.ds(j * G, G)], gsem))
      descs.append(pltpu.make_async_copy(
          xr_hbm.at[idx_d.at[j]], xj_v.at[pl.ds(j * G, G)], gsem))
    for d in descs:
      d.start()
    for d in descs:
      d.wait()

    def edge_body(e, c2):
      va = xi_v[e, pl.ds(0, L)] + xj_v[e, pl.ds(0, L)] + ea_v[e, pl.ds(0, L)]
      vb = xi_v[e, pl.ds(L, L)] + xj_v[e, pl.ds(L, L)] + ea_v[e, pl.ds(L, L)]
      va = jnp.maximum(va, va * 0.2)
      vb = jnp.maximum(vb, vb * 0.2)
      t = va * att_lo + vb * att_hi
      logit_v[e, pl.ds(0, L)] = plsc.cumsum(t)
      return c2
    lax.fori_loop(0, C, edge_body, 0)

    lane_iota = lax.iota(jnp.int32, L)
    last_lane = jnp.full((L,), L - 1, jnp.int32)
    for g in range(C // L):
      lg = plsc.load_gather(logit_v, [g * L + lane_iota, last_lane])
      pv = jnp.exp(lg)
      pbuf_v[pl.ds(ch * C + g * L, L)] = pv
      dd = idx_d[g // (G // L), pl.ds((g % (G // L)) * L, L)]
      plsc.addupdate_scatter(denom_v, [dd], pv)
    return c
  lax.fori_loop(0, NCH, chunk_body, 0)

  pltpu.sync_copy(pbuf_v, p_hbm.at[pl.ds(e0, EW)])
  pltpu.sync_copy(denom_v, dpart_hbm.at[wid])


def _sc_pass_a(src2d, dst2d, xl, xr, eap, att):
  kfn = pl.kernel(
      _pass_a_body,
      out_type=(jax.ShapeDtypeStruct((E,), jnp.float32),
                jax.ShapeDtypeStruct((NW, N), jnp.float32)),
      mesh=_sc_mesh(),
      compiler_params=pltpu.CompilerParams(needs_layout_passes=False, use_tc_tiling_on_sc=False),
      scratch_types=[
          pltpu.VMEM((RPC, G), jnp.int32),
          pltpu.VMEM((RPC, G), jnp.int32),
          pltpu.VMEM((C, H), jnp.float32),
          pltpu.VMEM((C, H), jnp.float32),
          pltpu.VMEM((C, H), jnp.float32),
          pltpu.VMEM((H,), jnp.float32),
          pltpu.VMEM((C, L), jnp.float32),
          pltpu.VMEM((EW,), jnp.float32),
          pltpu.VMEM((N,), jnp.float32),
          pltpu.SemaphoreType.DMA,
      ],
  )
  return kfn(src2d, dst2d, xl, xr, eap, att)


def _pass_b_body(src_hbm, dst_hbm, xl_hbm, p_hbm, rden_hbm,
                 alpha_hbm, opart_hbm,
                 idx_s, idx_d, xj_v, p_v, rden_v, abuf_v, zbuf_v,
                 acc_shared, gsem):
  cid = lax.axis_index("c")
  sid = lax.axis_index("s")
  wid = sid * NC + cid
  e0 = wid * EW

  pltpu.sync_copy(rden_hbm, rden_v)

  zero = jnp.zeros((L,), jnp.float32)

  def zrow(i, c):
    zbuf_v[i, pl.ds(0, L)] = zero
    zbuf_v[i, pl.ds(L, L)] = zero
    return c
  lax.fori_loop(0, ZB, zrow, 0)
  for k in range(STR // ZB):
    pltpu.sync_copy(zbuf_v, acc_shared.at[pl.ds(sid * STR + k * ZB, ZB)])
  @pl.when(sid == NS - 1)
  def _():
    pltpu.sync_copy(zbuf_v.at[pl.ds(0, TAIL)],
                    acc_shared.at[pl.ds(NS * STR, TAIL)])
  plsc.subcore_barrier()

  def chunk_body(ch, c):
    pltpu.sync_copy(src_hbm.at[wid, ch], idx_s)
    pltpu.sync_copy(dst_hbm.at[wid, ch], idx_d)
    pltpu.sync_copy(p_hbm.at[pl.ds(e0 + ch * C, C)], p_v)
    descs = []
    for j in range(RPC):
      descs.append(pltpu.make_async_copy(
          xl_hbm.at[idx_s.at[j]], xj_v.at[pl.ds(j * G, G)], gsem))
    for d in descs:
      d.start()
    for d in descs:
      d.wait()

    for g in range(C // L):
      dd = idx_d[g // (G // L), pl.ds((g % (G // L)) * L, L)]
      dv = plsc.load_gather(rden_v, [dd])
      al = p_v[pl.ds(g * L, L)] * dv
      abuf_v[pl.ds(ch * C + g * L, L)] = al

    def grp_body(g, c2):
      al = abuf_v[pl.ds(ch * C + g * L, L)]
      for k in range(L):
        e = g * L + k
        a_s = al[k]
        xj_v[e, pl.ds(0, L)] = xj_v[e, pl.ds(0, L)] * a_s
        xj_v[e, pl.ds(L, L)] = xj_v[e, pl.ds(L, L)] * a_s
      return c2
    lax.fori_loop(0, C // L, grp_body, 0)

    for j in range(RPC):
      pltpu.sync_copy(xj_v.at[pl.ds(j * G, G)], acc_shared.at[idx_d.at[j]],
                      add=True)
    return c
  lax.fori_loop(0, NCH, chunk_body, 0)

  plsc.subcore_barrier()
  pltpu.sync_copy(acc_shared.at[pl.ds(sid * STR, STR)],
                  opart_hbm.at[cid, pl.ds(sid * STR, STR)])
  @pl.when(sid == NS - 1)
  def _():
    pltpu.sync_copy(acc_shared.at[pl.ds(NS * STR, TAIL)],
                    opart_hbm.at[cid, pl.ds(NS * STR, TAIL)])
  pltpu.sync_copy(abuf_v, alpha_hbm.at[pl.ds(e0, EW)])


def _sc_pass_b(src2d, dst2d, xl, p, rden):
  kfn = pl.kernel(
      _pass_b_body,
      out_type=(jax.ShapeDtypeStruct((E,), jnp.float32),
                jax.ShapeDtypeStruct((NC, N, H), jnp.float32)),
      mesh=_sc_mesh(),
      compiler_params=pltpu.CompilerParams(needs_layout_passes=False, use_tc_tiling_on_sc=False),
      scratch_types=[
          pltpu.VMEM((RPC, G), jnp.int32),
          pltpu.VMEM((RPC, G), jnp.int32),
          pltpu.VMEM((C, H), jnp.float32),
          pltpu.VMEM((C,), jnp.float32),
          pltpu.VMEM((N,), jnp.float32),
          pltpu.VMEM((EW,), jnp.float32),
          pltpu.VMEM((ZB, H), jnp.float32),
          pltpu.VMEM_SHARED((N, H), jnp.float32),
          pltpu.SemaphoreType.DMA,
      ],
  )
  return kfn(src2d, dst2d, xl, p, rden)


def kernel(x, edge_index, edge_attr,
           W1l, W1r, b1l, b1r, att1, We1, bias1,
           W2l, W2r, b2l, b2r, att2, We2, bias2):
  idx4 = edge_index.reshape(2, NW, NCH, RPC, G)
  src2d = idx4[0]
  dst2d = idx4[1]

  xl1, xr1 = _node_proj(x, W1l, W1r, b1l, b1r)
  eap1 = _edge_proj(edge_attr, We1)
  p1, dpart1 = _sc_pass_a(src2d, dst2d, xl1, xr1, eap1, att1)
  rden1 = _denom_combine(dpart1).reshape(N)
  a1, opart1 = _sc_pass_b(src2d, dst2d, xl1, p1, rden1)

  xl2, xr2 = _assemble_proj(opart1, bias1, W2l, W2r, b2l, b2r)
  eap2 = _edge_mul(a1.reshape(E, 1), We2)
  p2, dpart2 = _sc_pass_a(src2d, dst2d, xl2, xr2, eap2, att2)
  rden2 = _denom_combine(dpart2).reshape(N)
  a2, opart2 = _sc_pass_b(src2d, dst2d, xl2, p2, rden2)

  x2 = _final_assemble(opart2, bias2)
  return (x2, edge_index, a2)


# speedup vs baseline: 11.0155x; 1.3866x over previous
import functools

import jax
import jax.numpy as jnp
from jax import lax
from jax.experimental import pallas as pl
from jax.experimental.pallas import tpu as pltpu
from jax.experimental.pallas import tpu_sc as plsc

N = 10000
E = 320000
D = 128
H = 32

NC = 2
NS = 16
NW = NC * NS
L = 16

EW = E // NW
C = 400
NCH = EW // C
G = 80
RPC = C // G
STR = 624
TAIL = N - NS * STR
ZB = 104


def _nodeproj_body(x_ref, wl_ref, wr_ref, bl_ref, br_ref, xl_ref, xr_ref):
  x = x_ref[...]
  xl_ref[...] = jnp.dot(x, wl_ref[...], preferred_element_type=jnp.float32) + bl_ref[...]
  xr_ref[...] = jnp.dot(x, wr_ref[...], preferred_element_type=jnp.float32) + br_ref[...]


def _node_proj(x, wl, wr, bl, br):
  return pl.pallas_call(
      _nodeproj_body,
      out_shape=(jax.ShapeDtypeStruct((N, H), jnp.float32),
                 jax.ShapeDtypeStruct((N, H), jnp.float32)),
  )(x, wl, wr, bl.reshape(1, H), br.reshape(1, H))


def _denom_body(dpart_ref, out_ref):
  s = jnp.sum(dpart_ref[...], axis=0, keepdims=True)
  out_ref[...] = 1.0 / (s + 1e-16)


def _denom_combine(dpart):
  return pl.pallas_call(
      _denom_body,
      out_shape=jax.ShapeDtypeStruct((1, N), jnp.float32),
  )(dpart)


def _assemble_body(op_ref, b_ref, wl_ref, wr_ref, bl_ref, br_ref,
                   xl_ref, xr_ref):
  h = jax.nn.relu(op_ref[0] + op_ref[1] + b_ref[...])
  xl_ref[...] = jnp.dot(h, wl_ref[...], preferred_element_type=jnp.float32) + bl_ref[...]
  xr_ref[...] = jnp.dot(h, wr_ref[...], preferred_element_type=jnp.float32) + br_ref[...]


def _assemble_proj(opart, bias, wl, wr, bl, br):
  return pl.pallas_call(
      _assemble_body,
      out_shape=(jax.ShapeDtypeStruct((N, H), jnp.float32),
                 jax.ShapeDtypeStruct((N, H), jnp.float32)),
  )(opart, bias.reshape(1, H), wl, wr, bl.reshape(1, H), br.reshape(1, H))


def _final_body(op_ref, b_ref, out_ref):
  out_ref[...] = op_ref[0] + op_ref[1] + b_ref[...]


def _final_assemble(opart, bias):
  return pl.pallas_call(
      _final_body,
      out_shape=jax.ShapeDtypeStruct((N, H), jnp.float32),
  )(opart, bias.reshape(1, H))


def _sc_mesh():
  return plsc.VectorSubcoreMesh(core_axis_name="c", subcore_axis_name="s",
                                num_cores=NC, num_subcores=NS)


def _make_pass_a_body(F):
  EPG = L // F

  def body(edge_hbm, ea_hbm, xl_hbm, xr_hbm, we_hbm, att_hbm,
           p_hbm, dpart_hbm,
           idx_s, idx_d, xi_v, xj_v, ea_v, we_v, att_v, logit_v, pbuf_v,
           denom_v, gsem):
    cid = lax.axis_index("c")
    sid = lax.axis_index("s")
    wid = sid * NC + cid
    e0 = wid * EW

    pltpu.sync_copy(att_hbm, att_v)
    pltpu.sync_copy(we_hbm, we_v)
    att_lo = att_v[pl.ds(0, L)]
    att_hi = att_v[pl.ds(L, L)]
    we_lo = [we_v[k, pl.ds(0, L)] for k in range(F)]
    we_hi = [we_v[k, pl.ds(L, L)] for k in range(F)]
    zero = jnp.zeros((L,), jnp.float32)

    def zero_body(i, c):
      denom_v[pl.ds(i * L, L)] = zero
      return c
    lax.fori_loop(0, N // L, zero_body, 0)

    def chunk_body(ch, c):
      eoff = e0 + ch * C
      pltpu.sync_copy(edge_hbm.at[0, pl.ds(eoff, C)], idx_s)
      pltpu.sync_copy(edge_hbm.at[1, pl.ds(eoff, C)], idx_d)
      pltpu.sync_copy(ea_hbm.at[pl.ds(eoff * F, C * F)], ea_v)
      descs = []
      for j in range(RPC):
        descs.append(pltpu.make_async_copy(
            xl_hbm.at[idx_s.at[pl.ds(j * G, G)]],
            xi_v.at[pl.ds(j * G, G)], gsem))
        descs.append(pltpu.make_async_copy(
            xr_hbm.at[idx_d.at[pl.ds(j * G, G)]],
            xj_v.at[pl.ds(j * G, G)], gsem))
      for d in descs:
        d.start()
      for d in descs:
        d.wait()

      def grp_body(g, c2):
        eav = ea_v[pl.ds(g * L, L)]
        for u in range(EPG):
          e = g * EPG + u
          va = xi_v[e, pl.ds(0, L)] + xj_v[e, pl.ds(0, L)]
          vb = xi_v[e, pl.ds(L, L)] + xj_v[e, pl.ds(L, L)]
          for k in range(F):
            sc = eav[u * F + k]
            va = va + sc * we_lo[k]
            vb = vb + sc * we_hi[k]
          va = jnp.maximum(va, va * 0.2)
          vb = jnp.maximum(vb, vb * 0.2)
          t = va * att_lo + vb * att_hi
          logit_v[e, pl.ds(0, L)] = plsc.cumsum(t)
        return c2
      lax.fori_loop(0, C // EPG, grp_body, 0)

      lane_iota = lax.iota(jnp.int32, L)
      last_lane = jnp.full((L,), L - 1, jnp.int32)
      for g in range(C // L):
        lg = plsc.load_gather(logit_v, [g * L + lane_iota, last_lane])
        pv = jnp.exp(lg)
        pbuf_v[pl.ds(ch * C + g * L, L)] = pv
        dd = idx_d[pl.ds(g * L, L)]
        plsc.addupdate_scatter(denom_v, [dd], pv)
      return c
    lax.fori_loop(0, NCH, chunk_body, 0)

    pltpu.sync_copy(pbuf_v, p_hbm.at[pl.ds(e0, EW)])
    pltpu.sync_copy(denom_v, dpart_hbm.at[wid])

  return body


def _sc_pass_a(edge_index, eaf, xl, xr, we, att):
  F = we.shape[0]
  kfn = pl.kernel(
      _make_pass_a_body(F),
      out_type=(jax.ShapeDtypeStruct((E,), jnp.float32),
                jax.ShapeDtypeStruct((NW, N), jnp.float32)),
      mesh=_sc_mesh(),
      compiler_params=pltpu.CompilerParams(needs_layout_passes=False, use_tc_tiling_on_sc=False),
      scratch_types=[
          pltpu.VMEM((C,), jnp.int32),
          pltpu.VMEM((C,), jnp.int32),
          pltpu.VMEM((C, H), jnp.float32),
          pltpu.VMEM((C, H), jnp.float32),
          pltpu.VMEM((C * F,), jnp.float32),
          pltpu.VMEM((F, H), jnp.float32),
          pltpu.VMEM((H,), jnp.float32),
          pltpu.VMEM((C, L), jnp.float32),
          pltpu.VMEM((EW,), jnp.float32),
          pltpu.VMEM((N,), jnp.float32),
          pltpu.SemaphoreType.DMA,
      ],
  )
  return kfn(edge_index, eaf, xl, xr, we, att)


def _pass_b_body(edge_hbm, xl_hbm, p_hbm, rden_hbm,
                 alpha_hbm, opart_hbm,
                 idx_s, idx_d, xj_v, p_v, rden_v, abuf_v, zbuf_v,
                 acc_shared, gsem):
  cid = lax.axis_index("c")
  sid = lax.axis_index("s")
  wid = sid * NC + cid
  e0 = wid * EW

  pltpu.sync_copy(rden_hbm, rden_v)

  zero = jnp.zeros((L,), jnp.float32)

  def zrow(i, c):
    zbuf_v[i, pl.ds(0, L)] = zero
    zbuf_v[i, pl.ds(L, L)] = zero
    return c
  lax.fori_loop(0, ZB, zrow, 0)
  for k in range(STR // ZB):
    pltpu.sync_copy(zbuf_v, acc_shared.at[pl.ds(sid * STR + k * ZB, ZB)])
  @pl.when(sid == NS - 1)
  def _():
    pltpu.sync_copy(zbuf_v.at[pl.ds(0, TAIL)],
                    acc_shared.at[pl.ds(NS * STR, TAIL)])
  plsc.subcore_barrier()

  def chunk_body(ch, c):
    eoff = e0 + ch * C
    pltpu.sync_copy(edge_hbm.at[0, pl.ds(eoff, C)], idx_s)
    for j in range(RPC):
      pltpu.sync_copy(edge_hbm.at[1, pl.ds(eoff + j * G, G)], idx_d.at[j])
    pltpu.sync_copy(p_hbm.at[pl.ds(eoff, C)], p_v)
    descs = []
    for j in range(RPC):
      descs.append(pltpu.make_async_copy(
          xl_hbm.at[idx_s.at[pl.ds(j * G, G)]],
          xj_v.at[pl.ds(j * G, G)], gsem))
    for d in descs:
      d.start()
    for d in descs:
      d.wait()

    for g in range(C // L):
      dd = idx_d[g // (G // L), pl.ds((g % (G // L)) * L, L)]
      dv = plsc.load_gather(rden_v, [dd])
      al = p_v[pl.ds(g * L, L)] * dv
      abuf_v[pl.ds(ch * C + g * L, L)] = al

    def grp_body(g, c2):
      al = abuf_v[pl.ds(ch * C + g * L, L)]
      for k in range(L):
        e = g * L + k
        a_s = al[k]
        xj_v[e, pl.ds(0, L)] = xj_v[e, pl.ds(0, L)] * a_s
        xj_v[e, pl.ds(L, L)] = xj_v[e, pl.ds(L, L)] * a_s
      return c2
    lax.fori_loop(0, C // L, grp_body, 0)

    for j in range(RPC):
      pltpu.sync_copy(xj_v.at[pl.ds(j * G, G)], acc_shared.at[idx_d.at[j]],
                      add=True)
    return c
  lax.fori_loop(0, NCH, chunk_body, 0)

  plsc.subcore_barrier()
  pltpu.sync_copy(acc_shared.at[pl.ds(sid * STR, STR)],
                  opart_hbm.at[cid, pl.ds(sid * STR, STR)])
  @pl.when(sid == NS - 1)
  def _():
    pltpu.sync_copy(acc_shared.at[pl.ds(NS * STR, TAIL)],
                    opart_hbm.at[cid, pl.ds(NS * STR, TAIL)])
  pltpu.sync_copy(abuf_v, alpha_hbm.at[pl.ds(e0, EW)])


def _sc_pass_b(edge_index, xl, p, rden):
  kfn = pl.kernel(
      _pass_b_body,
      out_type=(jax.ShapeDtypeStruct((E,), jnp.float32),
                jax.ShapeDtypeStruct((NC, N, H), jnp.float32)),
      mesh=_sc_mesh(),
      compiler_params=pltpu.CompilerParams(needs_layout_passes=False, use_tc_tiling_on_sc=False),
      scratch_types=[
          pltpu.VMEM((C,), jnp.int32),
          pltpu.VMEM((RPC, G), jnp.int32),
          pltpu.VMEM((C, H), jnp.float32),
          pltpu.VMEM((C,), jnp.float32),
          pltpu.VMEM((N,), jnp.float32),
          pltpu.VMEM((EW,), jnp.float32),
          pltpu.VMEM((ZB, H), jnp.float32),
          pltpu.VMEM_SHARED((N, H), jnp.float32),
          pltpu.SemaphoreType.DMA,
      ],
  )
  return kfn(edge_index, xl, p, rden)


def kernel(x, edge_index, edge_attr,
           W1l, W1r, b1l, b1r, att1, We1, bias1,
           W2l, W2r, b2l, b2r, att2, We2, bias2):
  xl1, xr1 = _node_proj(x, W1l, W1r, b1l, b1r)
  p1, dpart1 = _sc_pass_a(edge_index, edge_attr.reshape(E * 4), xl1, xr1,
                          We1, att1)
  rden1 = _denom_combine(dpart1).reshape(N)
  a1, opart1 = _sc_pass_b(edge_index, xl1, p1, rden1)

  xl2, xr2 = _assemble_proj(opart1, bias1, W2l, W2r, b2l, b2r)
  p2, dpart2 = _sc_pass_a(edge_index, a1, xl2, xr2, We2, att2)
  rden2 = _denom_combine(dpart2).reshape(N)
  a2, opart2 = _sc_pass_b(edge_index, xl2, p2, rden2)

  x2 = _final_assemble(opart2, bias2)
  return (x2, edge_index, a2)
